# baseline (speedup 1.0000x reference)
import jax
import jax.numpy as jnp
from jax import lax
from jax.experimental import pallas as pl
from jax.experimental.pallas import tpu as pltpu
from jax.experimental.pallas import tpu_sc as plsc

_MAXPOS = 77
_D = 768
_B = 4096
_L = 77
_NC = 2
_NS = 16
_NW = _NC * _NS
_SEQ_W = _B // _NW
_K = 32
_QW = _SEQ_W // _K
_NBUF = _QW
_PRE = 2
_NCHUNK = _L * _QW
_POSP = 16
_LANES = 16


def _sc_body(table_hbm, ids_hbm, pos_hbm, out_hbm, idx_v, pos_v,
             buf0, buf1, buf2, buf3,
             sg0, sg1, sg2, sg3,
             ss0, ss1, ss2, ss3):
    bufs = [buf0, buf1, buf2, buf3]
    sg = [sg0, sg1, sg2, sg3]
    ss = [ss0, ss1, ss2, ss3]
    wid = lax.axis_index("s") * _NC + lax.axis_index("c")
    b0 = wid * _SEQ_W

    def start_gather(c, j):
        pltpu.async_copy(
            table_hbm.at[idx_v.at[c // _QW, pl.ds((c % _QW) * _K, _K)]],
            bufs[j], sg[j],
        )

    def wait_gather(j):
        pltpu.make_async_copy(
            table_hbm.at[idx_v.at[0, pl.ds(0, _K)]], bufs[j], sg[j]
        ).wait()

    def start_store(c, j):
        pltpu.async_copy(
            bufs[j],
            out_hbm.at[c // _QW, pl.ds(b0 + (c % _QW) * _K, _K), :], ss[j],
        )

    def wait_store(c, j):
        pltpu.make_async_copy(
            bufs[j],
            out_hbm.at[c // _QW, pl.ds(b0 + (c % _QW) * _K, _K), :], ss[j],
        ).wait()

    pltpu.sync_copy(ids_hbm.at[:, pl.ds(b0, _SEQ_W)], idx_v)

    for c in range(_PRE):
        start_gather(c, c)

    def step(t, carry):
        @pl.when(t & (_POSP - 1) == 0)
        def _():
            off = pl.multiple_of(t & ~(_POSP - 1), _POSP)
            pltpu.sync_copy(pos_hbm.at[pl.ds(off, _POSP)], pos_v)

        for j in range(_NBUF):
            c = t * _NBUF + j
            j2 = (j + _PRE) % _NBUF

            @pl.when(c >= 2)
            def _():
                wait_store(c - 2, j2)

            @pl.when(c + _PRE < _NCHUNK)
            def _():
                start_gather(c + _PRE, j2)

            wait_gather(j)

            @plsc.parallel_loop(0, _D // _LANES, unroll=2)
            def _(k):
                v = pos_v[t & (_POSP - 1), pl.ds(k * _LANES, _LANES)]
                for i in range(_K):
                    plsc.addupdate(bufs[j].at[i, pl.ds(k * _LANES, _LANES)], v)

            start_store(c, j)
        return carry

    lax.fori_loop(0, _L, step, 0)
    wait_store(_NCHUNK - 2, (_NCHUNK - 2) % _NBUF)
    wait_store(_NCHUNK - 1, (_NCHUNK - 1) % _NBUF)


def kernel(input_ids, token_table, pos_table):
    ids_t = input_ids.T
    pos_pad = jnp.pad(pos_table, ((0, 80 - _MAXPOS), (0, 0)))

    mesh = plsc.VectorSubcoreMesh(core_axis_name="c", subcore_axis_name="s")
    run = pl.kernel(
        _sc_body,
        mesh=mesh,
        out_type=jax.ShapeDtypeStruct((_L, _B, _D), jnp.float32),
        compiler_params=pltpu.CompilerParams(use_tc_tiling_on_sc=True),
        scratch_types=(
            [
                pltpu.VMEM((_L, _SEQ_W), jnp.int32),
                pltpu.VMEM((_POSP, _D), jnp.float32),
            ]
            + [pltpu.VMEM((_K, _D), jnp.float32) for _ in range(_NBUF)]
            + [pltpu.SemaphoreType.DMA for _ in range(2 * _NBUF)]
        ),
    )
    out = run(token_table, ids_t, pos_pad)
    return out.transpose(1, 0, 2)


# speedup vs baseline: 1.0075x; 1.0075x over previous
import jax
import jax.numpy as jnp
from jax import lax
from jax.experimental import pallas as pl
from jax.experimental.pallas import tpu as pltpu
from jax.experimental.pallas import tpu_sc as plsc

_MAXPOS = 77
_D = 768
_B = 4096
_L = 77
_NC = 2
_NS = 16
_NW = _NC * _NS
_SEQ_W = _B // _NW
_K = 16
_QW = _SEQ_W // _K
_NBUF = 4
_NCHUNK = _L * _QW
_NSTEP = _NCHUNK // _NBUF
_LANES = 16


def _sc_body(table_hbm, ids_hbm, pos_hbm, out_hbm, idx_v, pos_v,
             buf0, buf1, buf2, buf3,
             sg0, sg1, sg2, sg3, ss0, ss1, ss2, ss3):
    bufs = [buf0, buf1, buf2, buf3]
    sg = [sg0, sg1, sg2, sg3]
    ss = [ss0, ss1, ss2, ss3]
    wid = lax.axis_index("s") * _NC + lax.axis_index("c")
    b0 = wid * _SEQ_W

    def start_gather(c, j):
        l = c // _QW
        q = c % _QW
        pltpu.async_copy(
            table_hbm.at[idx_v.at[l, pl.ds(q * _K, _K)]], bufs[j], sg[j]
        )

    def wait_gather(j):
        pltpu.make_async_copy(
            table_hbm.at[idx_v.at[0, pl.ds(0, _K)]], bufs[j], sg[j]
        ).wait()

    def start_store(c, j):
        l = c // _QW
        q = c % _QW
        pltpu.async_copy(
            bufs[j], out_hbm.at[l, pl.ds(b0 + q * _K, _K), :], ss[j]
        )

    def wait_store(c, j):
        l = c // _QW
        q = c % _QW
        pltpu.make_async_copy(
            bufs[j], out_hbm.at[l, pl.ds(b0 + q * _K, _K), :], ss[j]
        ).wait()

    pltpu.sync_copy(pos_hbm, pos_v)
    pltpu.sync_copy(ids_hbm.at[:, pl.ds(b0, _SEQ_W)], idx_v)

    start_gather(0, 0)
    start_gather(1, 1)

    def step(t, carry):
        for j in range(_NBUF):
            c = t * _NBUF + j
            j2 = (j + 2) % _NBUF

            @pl.when(c >= 2)
            def _():
                wait_store(c - 2, j2)

            @pl.when(c + 2 < _NCHUNK)
            def _():
                start_gather(c + 2, j2)

            wait_gather(j)
            l = c // _QW

            @plsc.parallel_loop(0, _D // _LANES, unroll=2)
            def _(k):
                v = pos_v[l, pl.ds(k * _LANES, _LANES)]
                for i in range(_K):
                    plsc.addupdate(bufs[j].at[i, pl.ds(k * _LANES, _LANES)], v)

            start_store(c, j)
        return carry

    lax.fori_loop(0, _NSTEP, step, 0)
    wait_store(_NCHUNK - 2, (_NCHUNK - 2) % _NBUF)
    wait_store(_NCHUNK - 1, (_NCHUNK - 1) % _NBUF)


def kernel(input_ids, token_table, pos_table):
    ids_t = input_ids.T

    mesh = plsc.VectorSubcoreMesh(core_axis_name="c", subcore_axis_name="s")
    run = pl.kernel(
        _sc_body,
        mesh=mesh,
        out_type=jax.ShapeDtypeStruct((_L, _B, _D), jnp.float32),
        compiler_params=pltpu.CompilerParams(use_tc_tiling_on_sc=True),
        scratch_types=[
            pltpu.VMEM((_L, _SEQ_W), jnp.int32),
            pltpu.VMEM((_MAXPOS, _D), jnp.float32),
            pltpu.VMEM((_K, _D), jnp.float32),
            pltpu.VMEM((_K, _D), jnp.float32),
            pltpu.VMEM((_K, _D), jnp.float32),
            pltpu.VMEM((_K, _D), jnp.float32),
            pltpu.SemaphoreType.DMA,
            pltpu.SemaphoreType.DMA,
            pltpu.SemaphoreType.DMA,
            pltpu.SemaphoreType.DMA,
            pltpu.SemaphoreType.DMA,
            pltpu.SemaphoreType.DMA,
            pltpu.SemaphoreType.DMA,
            pltpu.SemaphoreType.DMA,
        ],
    )
    out = run(token_table, ids_t, pos_table)
    return out.transpose(1, 0, 2)
